# baseline (speedup 1.0000x reference)
import functools

import jax
import jax.numpy as jnp
from jax import lax
from jax.experimental import pallas as pl
from jax.experimental.pallas import tpu as pltpu
from jax.experimental.pallas import tpu_sc as plsc

N = 10000
D = 128
DH = D // 2
E = 320000
NC = 2
NS = 16
NW = NC * NS
EPW = E // NW
EPT = E // NS
CHUNK = 80
NCHUNK = EPT // CHUNK
NBUF = 6
LAG = 3
NPAD = 10240
RPT = NPAD // NS
RLAST = N - (NS - 1) * RPT
BM = 2000

_sc_mesh = plsc.VectorSubcoreMesh(core_axis_name="c", subcore_axis_name="s")


@functools.partial(
    pl.kernel,
    out_type=jax.ShapeDtypeStruct((NW, NPAD), jnp.float32),
    mesh=_sc_mesh,
    scratch_types=[
        pltpu.VMEM((EPW,), jnp.int32),
        pltpu.VMEM((NPAD,), jnp.float32),
    ],
    compiler_params=pltpu.CompilerParams(needs_layout_passes=False),
)
def _deg_kernel(dst_hbm, hist_hbm, dst_v, hist_v):
    c = lax.axis_index("c")
    s = lax.axis_index("s")
    w = c * NS + s
    pltpu.sync_copy(dst_hbm.at[w], dst_v)
    zeros16 = jnp.zeros((16,), jnp.float32)

    def zbody(i, carry):
        hist_v[pl.ds(i * 16, 16)] = zeros16
        return carry

    lax.fori_loop(0, NPAD // 16, zbody, 0)
    ones16 = jnp.ones((16,), jnp.float32)

    def body(i, carry):
        v = dst_v[pl.ds(i * 16, 16)]
        plsc.addupdate_scatter(hist_v, [v], ones16)
        return carry

    lax.fori_loop(0, EPW // 16, body, 0)
    pltpu.sync_copy(hist_v, hist_hbm.at[w])


@functools.partial(
    pl.kernel,
    out_type=jax.ShapeDtypeStruct((NC, N, DH), jnp.float32),
    mesh=_sc_mesh,
    scratch_types=[
        pltpu.VMEM_SHARED((NPAD, DH), jnp.float32),
        pltpu.VMEM((NCHUNK, CHUNK), jnp.int32),
        pltpu.VMEM((NCHUNK, CHUNK), jnp.int32),
        pltpu.VMEM((CHUNK, DH), jnp.float32),
        pltpu.VMEM((CHUNK, DH), jnp.float32),
        pltpu.VMEM((CHUNK, DH), jnp.float32),
        pltpu.VMEM((CHUNK, DH), jnp.float32),
        pltpu.VMEM((CHUNK, DH), jnp.float32),
        pltpu.VMEM((CHUNK, DH), jnp.float32),
        pltpu.SemaphoreType.DMA,
        pltpu.SemaphoreType.DMA,
        pltpu.SemaphoreType.DMA,
        pltpu.SemaphoreType.DMA,
        pltpu.SemaphoreType.DMA,
        pltpu.SemaphoreType.DMA,
        pltpu.SemaphoreType.DMA,
        pltpu.SemaphoreType.DMA,
        pltpu.SemaphoreType.DMA,
        pltpu.SemaphoreType.DMA,
        pltpu.SemaphoreType.DMA,
        pltpu.SemaphoreType.DMA,
    ],
    compiler_params=pltpu.CompilerParams(use_tc_tiling_on_sc=False),
)
def _agg_kernel(u_hbm, src_hbm, dst_hbm, out_hbm, acc, src_v, dst_v,
                r0, r1, r2, r3, r4, r5,
                gs0, gs1, gs2, gs3, gs4, gs5,
                ss0, ss1, ss2, ss3, ss4, ss5):
    bufs = (r0, r1, r2, r3, r4, r5)
    gsems = (gs0, gs1, gs2, gs3, gs4, gs5)
    ssems = (ss0, ss1, ss2, ss3, ss4, ss5)
    c = lax.axis_index("c")
    s = lax.axis_index("s")
    uh = u_hbm.at[c]
    pltpu.sync_copy(src_hbm.at[s], src_v)
    pltpu.sync_copy(dst_hbm.at[s], dst_v)

    @pl.when(s < NS - 1)
    def _init_full():
        pltpu.sync_copy(uh.at[pl.ds(s * RPT, RPT)], acc.at[pl.ds(s * RPT, RPT)])

    @pl.when(s == NS - 1)
    def _init_last():
        pltpu.sync_copy(uh.at[pl.ds(s * RPT, RLAST)], acc.at[pl.ds(s * RPT, RLAST)])

    plsc.subcore_barrier()

    def fire_gather(j, b):
        pltpu.async_copy(uh.at[src_v.at[j]], bufs[b], gsems[b])

    def wait_gather(j, b):
        pltpu.make_async_copy(uh.at[src_v.at[j]], bufs[b], gsems[b]).wait()

    def fire_scatter(j, b):
        pltpu.async_copy(bufs[b], acc.at[dst_v.at[j]], ssems[b], add=True)

    def wait_scatter(j, b):
        pltpu.make_async_copy(bufs[b], acc.at[dst_v.at[j]], ssems[b]).wait()

    for b in range(NBUF):
        fire_gather(b, b)

    for j in range(NBUF):
        if j >= LAG:
            wait_scatter(j - LAG, (j - LAG) % NBUF)
            fire_gather(j - LAG + NBUF, (j - LAG) % NBUF)
        wait_gather(j, j)
        fire_scatter(j, j)

    def body(g, carry):
        for b in range(NBUF):
            j = g * NBUF + b
            wait_scatter(j - LAG, (b - LAG) % NBUF)
            fire_gather(j - LAG + NBUF, (b - LAG) % NBUF)
            wait_gather(j, b)
            fire_scatter(j, b)
        return carry

    TAIL = NBUF + NCHUNK % NBUF
    lax.fori_loop(1, (NCHUNK - TAIL) // NBUF, body, 0)

    jl = NCHUNK - TAIL
    for b in range(TAIL):
        j = jl + b
        wait_scatter(j - LAG, (j - LAG) % NBUF)
        if j - LAG + NBUF < NCHUNK:
            fire_gather(j - LAG + NBUF, (j - LAG) % NBUF)
        wait_gather(j, j % NBUF)
        fire_scatter(j, j % NBUF)
    for k in range(LAG):
        wait_scatter(NCHUNK - LAG + k, (NCHUNK - LAG + k) % NBUF)

    plsc.subcore_barrier()

    @pl.when(s < NS - 1)
    def _out_full():
        pltpu.sync_copy(acc.at[pl.ds(s * RPT, RPT)], out_hbm.at[c, pl.ds(s * RPT, RPT)])

    @pl.when(s == NS - 1)
    def _out_last():
        pltpu.sync_copy(acc.at[pl.ds(s * RPT, RLAST)], out_hbm.at[c, pl.ds(s * RPT, RLAST)])


def _dis_body(hist_ref, dis_ref):
    deg = jnp.sum(hist_ref[...], axis=0) + 1.0
    dis_ref[...] = lax.rsqrt(deg)


def _dis_call(hist):
    return pl.pallas_call(
        _dis_body,
        out_shape=jax.ShapeDtypeStruct((NPAD // 128, 128), jnp.float32),
    )(hist)


def _mm_scale_body(x_ref, w_ref, dis_ref, o_ref):
    h = jnp.dot(x_ref[...], w_ref[...], preferred_element_type=jnp.float32,
                precision=lax.Precision.HIGHEST)
    u = h * dis_ref[...]
    o_ref[0] = u[:, :DH]
    o_ref[1] = u[:, DH:]


def _mm_scale(x, w, dis_col):
    return pl.pallas_call(
        _mm_scale_body,
        grid=(N // BM,),
        in_specs=[
            pl.BlockSpec((BM, D), lambda i: (i, 0)),
            pl.BlockSpec((D, D), lambda i: (0, 0)),
            pl.BlockSpec((BM, 1), lambda i: (i, 0)),
        ],
        out_specs=pl.BlockSpec((NC, BM, DH), lambda i: (0, i, 0)),
        out_shape=jax.ShapeDtypeStruct((NC, N, DH), jnp.float32),
    )(x, w, dis_col)


def _mid_body(p_ref, dis_ref, b_ref, w_ref, o_ref):
    agg = jnp.concatenate([p_ref[0], p_ref[1]], axis=1)
    x2 = jnp.maximum(agg * dis_ref[...] + b_ref[...], 0.0)
    h2 = jnp.dot(x2, w_ref[...], preferred_element_type=jnp.float32,
                 precision=lax.Precision.HIGHEST)
    u2 = h2 * dis_ref[...]
    o_ref[0] = u2[:, :DH]
    o_ref[1] = u2[:, DH:]


def _mid(p, dis_col, b, w):
    return pl.pallas_call(
        _mid_body,
        grid=(N // BM,),
        in_specs=[
            pl.BlockSpec((NC, BM, DH), lambda i: (0, i, 0)),
            pl.BlockSpec((BM, 1), lambda i: (i, 0)),
            pl.BlockSpec((1, D), lambda i: (0, 0)),
            pl.BlockSpec((D, D), lambda i: (0, 0)),
        ],
        out_specs=pl.BlockSpec((NC, BM, DH), lambda i: (0, i, 0)),
        out_shape=jax.ShapeDtypeStruct((NC, N, DH), jnp.float32),
    )(p, dis_col, b, w)


def _final_body(p_ref, dis_ref, b_ref, o_ref):
    agg = jnp.concatenate([p_ref[0], p_ref[1]], axis=1)
    o_ref[...] = jnp.maximum(agg * dis_ref[...] + b_ref[...], 0.0)


def _final(p, dis_col, b):
    return pl.pallas_call(
        _final_body,
        grid=(N // BM,),
        in_specs=[
            pl.BlockSpec((NC, BM, DH), lambda i: (0, i, 0)),
            pl.BlockSpec((BM, 1), lambda i: (i, 0)),
            pl.BlockSpec((1, D), lambda i: (0, 0)),
        ],
        out_specs=pl.BlockSpec((BM, D), lambda i: (i, 0)),
        out_shape=jax.ShapeDtypeStruct((N, D), jnp.float32),
    )(p, dis_col, b)


def kernel(node_fts, edge_index, W1, b1, W2, b2):
    ei = edge_index.astype(jnp.int32)
    srcH = ei[0].reshape(NS, NCHUNK, CHUNK)
    dstH = ei[1].reshape(NS, NCHUNK, CHUNK)
    dstw = ei[1].reshape(NW, EPW)

    hist = _deg_kernel(dstw)
    dis = _dis_call(hist.reshape(NW, NPAD // 128, 128))
    dis_col = dis.reshape(NPAD)[:N].reshape(N, 1)

    u1 = _mm_scale(node_fts, W1, dis_col)
    p1 = _agg_kernel(u1, srcH, dstH)
    u2 = _mid(p1, dis_col, b1.reshape(1, D), W2)
    p2 = _agg_kernel(u2, srcH, dstH)
    return _final(p2, dis_col, b2.reshape(1, D))


# speedup vs baseline: 1.0757x; 1.0757x over previous
import functools

import jax
import jax.numpy as jnp
from jax import lax
from jax.experimental import pallas as pl
from jax.experimental.pallas import tpu as pltpu
from jax.experimental.pallas import tpu_sc as plsc

N = 10000
D = 128
E = 320000
NC = 2
NS = 16
NW = NC * NS
EPW = E // NW
CHUNK = 80
NCHUNK = EPW // CHUNK
NBUF = 3
LAG = 2
NPAD = 10240
RPT = NPAD // NS
RLAST = N - (NS - 1) * RPT
BM = 2000

_sc_mesh = plsc.VectorSubcoreMesh(core_axis_name="c", subcore_axis_name="s")


@functools.partial(
    pl.kernel,
    out_type=jax.ShapeDtypeStruct((NW, NPAD), jnp.float32),
    mesh=_sc_mesh,
    scratch_types=[
        pltpu.VMEM((EPW,), jnp.int32),
        pltpu.VMEM((NPAD,), jnp.float32),
    ],
    compiler_params=pltpu.CompilerParams(needs_layout_passes=False),
)
def _deg_kernel(dst_hbm, hist_hbm, dst_v, hist_v):
    c = lax.axis_index("c")
    s = lax.axis_index("s")
    w = c * NS + s
    pltpu.sync_copy(dst_hbm.at[w], dst_v)
    zeros16 = jnp.zeros((16,), jnp.float32)

    def zbody(i, carry):
        hist_v[pl.ds(i * 16, 16)] = zeros16
        return carry

    lax.fori_loop(0, NPAD // 16, zbody, 0)
    ones16 = jnp.ones((16,), jnp.float32)

    def body(i, carry):
        v = dst_v[pl.ds(i * 16, 16)]
        plsc.addupdate_scatter(hist_v, [v], ones16)
        return carry

    lax.fori_loop(0, EPW // 16, body, 0)
    pltpu.sync_copy(hist_v, hist_hbm.at[w])


@functools.partial(
    pl.kernel,
    out_type=jax.ShapeDtypeStruct((NC, N, D), jnp.float32),
    mesh=_sc_mesh,
    scratch_types=[
        pltpu.VMEM_SHARED((N, D), jnp.float32),
        pltpu.VMEM((NCHUNK, CHUNK), jnp.int32),
        pltpu.VMEM((NCHUNK, CHUNK), jnp.int32),
        pltpu.VMEM((CHUNK, D), jnp.float32),
        pltpu.VMEM((CHUNK, D), jnp.float32),
        pltpu.VMEM((CHUNK, D), jnp.float32),
        pltpu.SemaphoreType.DMA,
        pltpu.SemaphoreType.DMA,
        pltpu.SemaphoreType.DMA,
        pltpu.SemaphoreType.DMA,
        pltpu.SemaphoreType.DMA,
        pltpu.SemaphoreType.DMA,
    ],
    compiler_params=pltpu.CompilerParams(use_tc_tiling_on_sc=False),
)
def _agg_kernel(u_hbm, src_hbm, dst_hbm, out_hbm, acc, src_v, dst_v,
                r0, r1, r2, gs0, gs1, gs2, ss0, ss1, ss2):
    bufs = (r0, r1, r2)
    gsems = (gs0, gs1, gs2)
    ssems = (ss0, ss1, ss2)
    c = lax.axis_index("c")
    s = lax.axis_index("s")
    w = c * NS + s
    pltpu.sync_copy(src_hbm.at[w], src_v)
    pltpu.sync_copy(dst_hbm.at[w], dst_v)
    @pl.when(s < NS - 1)
    def _init_full():
        pltpu.sync_copy(u_hbm.at[pl.ds(s * RPT, RPT)], acc.at[pl.ds(s * RPT, RPT)])

    @pl.when(s == NS - 1)
    def _init_last():
        pltpu.sync_copy(u_hbm.at[pl.ds(s * RPT, RLAST)], acc.at[pl.ds(s * RPT, RLAST)])

    plsc.subcore_barrier()

    def fire_gather(j, b):
        pltpu.async_copy(u_hbm.at[src_v.at[j]], bufs[b], gsems[b])

    def wait_gather(j, b):
        pltpu.make_async_copy(u_hbm.at[src_v.at[j]], bufs[b], gsems[b]).wait()

    def fire_scatter(j, b):
        pltpu.async_copy(bufs[b], acc.at[dst_v.at[j]], ssems[b], add=True)

    def wait_scatter(j, b):
        pltpu.make_async_copy(bufs[b], acc.at[dst_v.at[j]], ssems[b]).wait()

    for b in range(NBUF):
        fire_gather(b, b)

    for j in range(NBUF):
        if j >= LAG:
            wait_scatter(j - LAG, (j - LAG) % NBUF)
            fire_gather(j - LAG + NBUF, (j - LAG) % NBUF)
        wait_gather(j, j)
        fire_scatter(j, j)

    def body(g, carry):
        for b in range(NBUF):
            j = g * NBUF + b
            wait_scatter(j - LAG, (b - LAG) % NBUF)
            fire_gather(j - LAG + NBUF, (b - LAG) % NBUF)
            wait_gather(j, b)
            fire_scatter(j, b)
        return carry

    TAIL = NBUF + NCHUNK % NBUF
    lax.fori_loop(1, (NCHUNK - TAIL) // NBUF, body, 0)

    jl = NCHUNK - TAIL
    for b in range(TAIL):
        j = jl + b
        wait_scatter(j - LAG, (j - LAG) % NBUF)
        if j - LAG + NBUF < NCHUNK:
            fire_gather(j - LAG + NBUF, (j - LAG) % NBUF)
        wait_gather(j, j % NBUF)
        fire_scatter(j, j % NBUF)
    for k in range(LAG):
        wait_scatter(NCHUNK - LAG + k, (NCHUNK - LAG + k) % NBUF)

    plsc.subcore_barrier()

    @pl.when(s < NS - 1)
    def _out_full():
        pltpu.sync_copy(acc.at[pl.ds(s * RPT, RPT)], out_hbm.at[c, pl.ds(s * RPT, RPT)])

    @pl.when(s == NS - 1)
    def _out_last():
        pltpu.sync_copy(acc.at[pl.ds(s * RPT, RLAST)], out_hbm.at[c, pl.ds(s * RPT, RLAST)])


def _dis_body(hist_ref, dis_ref):
    deg = jnp.sum(hist_ref[...], axis=0) + 1.0
    dis_ref[...] = lax.rsqrt(deg)


def _dis_call(hist):
    return pl.pallas_call(
        _dis_body,
        out_shape=jax.ShapeDtypeStruct((NPAD // 128, 128), jnp.float32),
    )(hist)


def _mm_scale_body(x_ref, w_ref, dis_ref, o_ref):
    h = jnp.dot(x_ref[...], w_ref[...], preferred_element_type=jnp.float32,
                precision=lax.Precision.HIGHEST)
    o_ref[...] = h * dis_ref[...]


def _mm_scale(x, w, dis_col):
    return pl.pallas_call(
        _mm_scale_body,
        grid=(N // BM,),
        in_specs=[
            pl.BlockSpec((BM, D), lambda i: (i, 0)),
            pl.BlockSpec((D, D), lambda i: (0, 0)),
            pl.BlockSpec((BM, 1), lambda i: (i, 0)),
        ],
        out_specs=pl.BlockSpec((BM, D), lambda i: (i, 0)),
        out_shape=jax.ShapeDtypeStruct((N, D), jnp.float32),
    )(x, w, dis_col)


def _mid_body(p_ref, u_ref, dis_ref, b_ref, w_ref, o_ref):
    x2 = jnp.maximum((p_ref[0] + p_ref[1] - u_ref[...]) * dis_ref[...]
                     + b_ref[...], 0.0)
    h2 = jnp.dot(x2, w_ref[...], preferred_element_type=jnp.float32,
                 precision=lax.Precision.HIGHEST)
    o_ref[...] = h2 * dis_ref[...]


def _mid(p, u, dis_col, b, w):
    return pl.pallas_call(
        _mid_body,
        grid=(N // BM,),
        in_specs=[
            pl.BlockSpec((NC, BM, D), lambda i: (0, i, 0)),
            pl.BlockSpec((BM, D), lambda i: (i, 0)),
            pl.BlockSpec((BM, 1), lambda i: (i, 0)),
            pl.BlockSpec((1, D), lambda i: (0, 0)),
            pl.BlockSpec((D, D), lambda i: (0, 0)),
        ],
        out_specs=pl.BlockSpec((BM, D), lambda i: (i, 0)),
        out_shape=jax.ShapeDtypeStruct((N, D), jnp.float32),
    )(p, u, dis_col, b, w)


def _final_body(p_ref, u_ref, dis_ref, b_ref, o_ref):
    o_ref[...] = jnp.maximum((p_ref[0] + p_ref[1] - u_ref[...]) * dis_ref[...]
                             + b_ref[...], 0.0)


def _final(p, u, dis_col, b):
    return pl.pallas_call(
        _final_body,
        grid=(N // BM,),
        in_specs=[
            pl.BlockSpec((NC, BM, D), lambda i: (0, i, 0)),
            pl.BlockSpec((BM, D), lambda i: (i, 0)),
            pl.BlockSpec((BM, 1), lambda i: (i, 0)),
            pl.BlockSpec((1, D), lambda i: (0, 0)),
        ],
        out_specs=pl.BlockSpec((BM, D), lambda i: (i, 0)),
        out_shape=jax.ShapeDtypeStruct((N, D), jnp.float32),
    )(p, u, dis_col, b)


def kernel(node_fts, edge_index, W1, b1, W2, b2):
    ei = edge_index.astype(jnp.int32)
    src3 = ei[0].reshape(NW, NCHUNK, CHUNK)
    dst3 = ei[1].reshape(NW, NCHUNK, CHUNK)
    dstw = ei[1].reshape(NW, EPW)

    hist = _deg_kernel(dstw)
    dis = _dis_call(hist.reshape(NW, NPAD // 128, 128))
    dis_col = dis.reshape(NPAD)[:N].reshape(N, 1)

    u1 = _mm_scale(node_fts, W1, dis_col)
    p1 = _agg_kernel(u1, src3, dst3)
    u2 = _mid(p1, u1, dis_col, b1.reshape(1, D), W2)
    p2 = _agg_kernel(u2, src3, dst3)
    return _final(p2, u2, dis_col, b2.reshape(1, D))
